# baseline (speedup 1.0000x reference)
import functools

import jax
import jax.numpy as jnp
from jax import lax
from jax.experimental import pallas as pl
from jax.experimental.pallas import tpu as pltpu
from jax.experimental.pallas import tpu_sc as plsc

B = 4096
L = 50
D = 64
V = 1_000_000
NTILE = V // 8
N = B * L
NC = 2
NS = 16
NW = NC * NS

RT = 16
W_TILES = 3_904
NCH_R = W_TILES // RT
NCH_R_LAST = 249

B_PER_W = N // NW
C = 128
NCH_G = B_PER_W // C

_mesh = plsc.VectorSubcoreMesh(core_axis_name="c", subcore_axis_name="s")


@functools.partial(
    pl.kernel,
    mesh=_mesh,
    out_type=jax.ShapeDtypeStruct((V, 128), jnp.float32),
    scratch_types=[
        pltpu.VMEM((RT, 8, D), jnp.float32),
        pltpu.VMEM((RT, 8, D), jnp.float32),
        pltpu.VMEM((8 * RT, 128), jnp.float32),
        pltpu.VMEM((8 * RT, 128), jnp.float32),
        pltpu.SemaphoreType.DMA,
        pltpu.SemaphoreType.DMA,
        pltpu.SemaphoreType.DMA,
        pltpu.SemaphoreType.DMA,
    ],
)
def _repack_kernel(t3_hbm, tp_hbm, tb0, tb1, pb0, pb1, g0, g1, w0, w1):
    wid = lax.axis_index("s") * NC + lax.axis_index("c")
    base = wid * W_TILES
    tb = (tb0, tb1)
    pb = (pb0, pb1)
    gsem = (g0, g1)
    wsem = (w0, w1)
    nch = lax.select(wid == NW - 1, jnp.int32(NCH_R_LAST), jnp.int32(NCH_R))

    def tile_start(j):
        return lax.min(base + j * RT, jnp.int32(NTILE - RT))

    def fetch(j, b):
        return pltpu.async_copy(
            t3_hbm.at[pl.ds(tile_start(j), RT)], tb[b], gsem[b])

    def repack(b):
        for k in range(RT):
            for s in range(8):
                for c in range(D // 16):
                    sl = pl.ds(16 * c, 16)
                    pb[b][8 * k + s, sl] = tb[b][k, s, sl]

    def flush(j, b):
        return pltpu.async_copy(
            pb[b], tp_hbm.at[pl.ds(tile_start(j) * 8, 8 * RT)], wsem[b])

    fetch(0, 0)

    def body(j, carry):
        for par in range(2):
            @pl.when(lax.rem(j, 2) == par)
            def _():
                cur, nxt = par, 1 - par

                @pl.when(j + 1 < nch)
                def _():
                    @pl.when(j >= 1)
                    def _():
                        pltpu.make_async_copy(
                            pb[nxt],
                            tp_hbm.at[pl.ds(0, 8 * RT)], wsem[nxt]).wait()
                    fetch(j + 1, nxt)

                pltpu.make_async_copy(
                    t3_hbm.at[pl.ds(0, RT)], tb[cur], gsem[cur]).wait()
                repack(cur)
                flush(j, cur)
        return carry

    lax.fori_loop(0, nch, body, 0)
    for par in range(2):
        @pl.when(lax.rem(nch, 2) == par)
        def _():
            pltpu.make_async_copy(
                pb[par],
                tp_hbm.at[pl.ds(0, 8 * RT)], wsem[par]).wait()
            pltpu.make_async_copy(
                pb[1 - par],
                tp_hbm.at[pl.ds(0, 8 * RT)], wsem[1 - par]).wait()


@functools.partial(
    pl.kernel,
    mesh=_mesh,
    out_type=jax.ShapeDtypeStruct((N, D), jnp.float32),
    scratch_types=[
        pltpu.VMEM((C,), jnp.int32),
        pltpu.VMEM((C,), jnp.int32),
        pltpu.VMEM((C, 128), jnp.float32),
        pltpu.VMEM((C, 128), jnp.float32),
        pltpu.VMEM((C, D), jnp.float32),
        pltpu.VMEM((C, D), jnp.float32),
        pltpu.SemaphoreType.DMA,
        pltpu.SemaphoreType.DMA,
        pltpu.SemaphoreType.DMA,
        pltpu.SemaphoreType.DMA,
    ],
)
def _gather_kernel(idx_hbm, tp_hbm, out_hbm,
                   xv0, xv1, pad0, pad1, rows0, rows1, g0, g1, w0, w1):
    wid = lax.axis_index("s") * NC + lax.axis_index("c")
    base = wid * B_PER_W
    xv = (xv0, xv1)
    pad = (pad0, pad1)
    rows = (rows0, rows1)
    gsem = (g0, g1)
    wsem = (w0, w1)

    def stage(g, b):
        pltpu.sync_copy(idx_hbm.at[pl.ds(base + g * C, C)], xv[b])
        pltpu.async_copy(tp_hbm.at[xv[b]], pad[b], gsem[b])

    def extract(b):
        for k in range(C):
            for c in range(D // 16):
                sl = pl.ds(16 * c, 16)
                rows[b][k, sl] = pad[b][k, sl]

    def writeback(g, b):
        return pltpu.async_copy(
            rows[b], out_hbm.at[pl.ds(base + g * C, C)], wsem[b])

    stage(0, 0)

    def body(t, carry):
        for par in range(2):
            @pl.when(lax.rem(t, 2) == par)
            def _():
                cur, nxt = par, 1 - par

                @pl.when(t + 1 < NCH_G)
                def _():
                    @pl.when(t >= 1)
                    def _():
                        pltpu.make_async_copy(
                            rows[nxt], out_hbm.at[pl.ds(0, C)],
                            wsem[nxt]).wait()
                    stage(t + 1, nxt)

                pltpu.make_async_copy(
                    tp_hbm.at[xv[cur]], pad[cur], gsem[cur]).wait()
                extract(cur)
                writeback(t, cur)
        return carry

    lax.fori_loop(0, NCH_G, body, 0)
    pltpu.make_async_copy(rows[(NCH_G - 2) % 2],
                          out_hbm.at[pl.ds(0, C)],
                          wsem[(NCH_G - 2) % 2]).wait()
    pltpu.make_async_copy(rows[(NCH_G - 1) % 2],
                          out_hbm.at[pl.ds(0, C)],
                          wsem[(NCH_G - 1) % 2]).wait()


def kernel(x, table):
    flat = x.reshape(N)
    t3 = table.reshape(NTILE, 8, D)
    tp = _repack_kernel(t3)
    out = _gather_kernel(flat, tp)
    return out.reshape(B, L, D)
